# Initial kernel scaffold; baseline (speedup 1.0000x reference)
import jax
import jax.numpy as jnp
from jax.experimental import pallas as pl


def kernel(token_ids, weight):
    raise NotImplementedError("write your pallas kernel here")


# speedup vs baseline: 1.1107x; 1.1107x over previous
import functools

import jax
import jax.numpy as jnp
from jax import lax
from jax.experimental import pallas as pl
from jax.experimental.pallas import tpu as pltpu
from jax.experimental.pallas import tpu_sc as plsc

NUM_EMB = 1_000_000
DIM = 32

NC, NS, LANES = 2, 16, 16
NW = NC * NS

B = 16384 * 50
IDXV = 128
ROWS_PER_W = B // NW
VECS_PER_W = ROWS_PER_W // IDXV
CH_VECS = 20
CH_ROWS = CH_VECS * IDXV
N_CHUNKS = VECS_PER_W // CH_VECS

_mesh = plsc.VectorSubcoreMesh(core_axis_name="c", subcore_axis_name="s")


@functools.partial(
    pl.kernel,
    out_type=jax.ShapeDtypeStruct((B, DIM), jnp.float32),
    mesh=_mesh,
    scratch_types=[
        pltpu.VMEM((VECS_PER_W, IDXV), jnp.int32),
        pltpu.VMEM((CH_ROWS, DIM), jnp.float32),
        pltpu.SemaphoreType.DMA,
    ],
    compiler_params=pltpu.CompilerParams(use_tc_tiling_on_sc=False),
)
def _emb_gather(idx_hbm, table_hbm, out_hbm, idx_v, rows_v, sem):
    wid = lax.axis_index("s") * NC + lax.axis_index("c")
    vec_base = wid * VECS_PER_W
    row_base = wid * ROWS_PER_W

    pltpu.sync_copy(idx_hbm.at[pl.ds(vec_base, VECS_PER_W)], idx_v)

    def chunk(g, _):
        copies = []
        for j in range(CH_VECS):
            c = pltpu.async_copy(
                table_hbm.at[idx_v.at[g * CH_VECS + j]],
                rows_v.at[pl.ds(j * IDXV, IDXV)],
                sem,
            )
            copies.append(c)
        for c in copies:
            c.wait()
        pltpu.sync_copy(
            rows_v, out_hbm.at[pl.ds(row_base + g * CH_ROWS, CH_ROWS)]
        )
        return 0

    lax.fori_loop(0, N_CHUNKS, chunk, 0)


def kernel(token_ids, weight):
    flat_idx = token_ids.reshape(B // IDXV, IDXV).astype(jnp.int32)
    out = _emb_gather(flat_idx, weight)
    return out.reshape(token_ids.shape + (DIM,))


# speedup vs baseline: 1.1129x; 1.0020x over previous
import functools

import jax
import jax.numpy as jnp
from jax import lax
from jax.experimental import pallas as pl
from jax.experimental.pallas import tpu as pltpu
from jax.experimental.pallas import tpu_sc as plsc

NUM_EMB = 1_000_000
DIM = 32

NC, NS, LANES = 2, 16, 16
NW = NC * NS

B = 16384 * 50
ROWS_PER_W = B // NW
CH_ROWS = 3200
N_CHUNKS = ROWS_PER_W // CH_ROWS

_mesh = plsc.VectorSubcoreMesh(core_axis_name="c", subcore_axis_name="s")


@functools.partial(
    pl.kernel,
    out_type=jax.ShapeDtypeStruct((B, DIM), jnp.float32),
    mesh=_mesh,
    scratch_types=[
        pltpu.VMEM((ROWS_PER_W,), jnp.int32),
        pltpu.VMEM((CH_ROWS, DIM), jnp.float32),
        pltpu.SemaphoreType.DMA,
    ],
    compiler_params=pltpu.CompilerParams(use_tc_tiling_on_sc=False),
)
def _emb_gather(idx_hbm, table_hbm, out_hbm, idx_v, rows_v, sem):
    wid = lax.axis_index("s") * NC + lax.axis_index("c")
    row_base = wid * ROWS_PER_W

    pltpu.sync_copy(idx_hbm.at[pl.ds(row_base, ROWS_PER_W)], idx_v)

    def chunk(g, _):
        pltpu.async_copy(
            table_hbm.at[idx_v.at[pl.ds(g * CH_ROWS, CH_ROWS)]],
            rows_v,
            sem,
        ).wait()
        pltpu.sync_copy(
            rows_v, out_hbm.at[pl.ds(row_base + g * CH_ROWS, CH_ROWS)]
        )
        return 0

    lax.fori_loop(0, N_CHUNKS, chunk, 0)


def kernel(token_ids, weight):
    flat_idx = token_ids.reshape(B).astype(jnp.int32)
    out = _emb_gather(flat_idx, weight)
    return out.reshape(token_ids.shape + (DIM,))


# speedup vs baseline: 1.5013x; 1.3490x over previous
import functools

import jax
import jax.numpy as jnp
from jax import lax
from jax.experimental import pallas as pl
from jax.experimental.pallas import tpu as pltpu
from jax.experimental.pallas import tpu_sc as plsc

NUM_EMB = 1_000_000
D = 32
NI = 16384
NJ = 50

NC, NS, LANES = 2, 16, 16
NW = NC * NS
IW = NI // NW
NPAIR = NJ // 2

_mesh = plsc.VectorSubcoreMesh(core_axis_name="c", subcore_axis_name="s")


@functools.partial(
    pl.kernel,
    out_type=jax.ShapeDtypeStruct((NJ, D, NI), jnp.float32),
    mesh=_mesh,
    scratch_types=[
        pltpu.VMEM((NJ, IW), jnp.int32),
        pltpu.VMEM((2, IW, D), jnp.float32),
        pltpu.VMEM((2, D, IW), jnp.float32),
        pltpu.SemaphoreType.DMA,
        pltpu.SemaphoreType.DMA,
        pltpu.SemaphoreType.DMA,
        pltpu.SemaphoreType.DMA,
    ],
    compiler_params=pltpu.CompilerParams(
        use_tc_tiling_on_sc=False, needs_layout_passes=False
    ),
)
def _emb_gather_t(tok_hbm, w_hbm, out_hbm, idx_v, gbuf, tbuf, sg0, sg1, ss0, ss1):
    wid = lax.axis_index("s") * NC + lax.axis_index("c")
    i0 = wid * IW
    iota = lax.iota(jnp.int32, LANES)

    pltpu.sync_copy(tok_hbm.at[:, pl.ds(i0, IW)], idx_v)

    def fire_gather(j, b, sem):
        pltpu.async_copy(w_hbm.at[idx_v.at[j]], gbuf.at[b], sem)

    def wait_gather(b, sem):
        pltpu.make_async_copy(w_hbm.at[idx_v.at[0]], gbuf.at[b], sem).wait()

    def fire_store(j, b, sem):
        pltpu.async_copy(tbuf.at[b], out_hbm.at[j, :, pl.ds(i0, IW)], sem)

    def wait_store(b, sem):
        pltpu.make_async_copy(
            tbuf.at[b], out_hbm.at[0, :, pl.ds(i0, IW)], sem
        ).wait()

    def transpose(b):
        g2, t2 = gbuf.at[b], tbuf.at[b]

        def per_d(d, _):
            colv = jnp.full((LANES,), d, jnp.int32)
            for g in range(IW // LANES):
                rows = g * LANES + iota
                vals = plsc.load_gather(g2, [rows, colv])
                t2[d, pl.ds(g * LANES, LANES)] = vals
            return 0

        lax.fori_loop(0, D, per_d, 0)

    fire_gather(0, 0, sg0)

    def pair(p, _):
        c0 = 2 * p
        fire_gather(c0 + 1, 1, sg1)
        wait_gather(0, sg0)

        @pl.when(p > 0)
        def _():
            wait_store(0, ss0)

        transpose(0)
        fire_store(c0, 0, ss0)

        @pl.when(p < NPAIR - 1)
        def _():
            fire_gather(c0 + 2, 0, sg0)

        wait_gather(1, sg1)

        @pl.when(p > 0)
        def _():
            wait_store(1, ss1)

        transpose(1)
        fire_store(c0 + 1, 1, ss1)
        return 0

    lax.fori_loop(0, NPAIR, pair, 0)
    wait_store(0, ss0)
    wait_store(1, ss1)


def kernel(token_ids, weight):
    tok_t = token_ids.T.astype(jnp.int32)
    out_t = _emb_gather_t(tok_t, weight)
    return jnp.transpose(out_t, (2, 0, 1))
